# Initial kernel scaffold; baseline (speedup 1.0000x reference)
import jax
import jax.numpy as jnp
from jax.experimental import pallas as pl


def kernel(x, sadj, fadj, fadj2, params):
    raise NotImplementedError("write your pallas kernel here")


# speedup vs baseline: 1.7596x; 1.7596x over previous
import functools

import jax
import jax.numpy as jnp
from jax.experimental import pallas as pl
from jax.experimental.pallas import tpu as pltpu


def _adjmm_kernel(a_ref, u_ref, b_ref, o_ref, *, rownorm_half):
    p = jnp.dot(a_ref[...], u_ref[...], preferred_element_type=jnp.float32)
    p = p + b_ref[...]
    if rownorm_half:
        h = p.shape[-1] // 2
        p1, p2 = p[:, :h], p[:, h:]
        n1 = jnp.maximum(jnp.sqrt(jnp.sum(p1 * p1, axis=1, keepdims=True)), 1e-12)
        n2 = jnp.maximum(jnp.sqrt(jnp.sum(p2 * p2, axis=1, keepdims=True)), 1e-12)
        p = jnp.concatenate([p1 / n1, p2 / n2], axis=1)
    o_ref[...] = p


def _adjmm(adj, u, b, rownorm_half=False, bm=200):
    n = adj.shape[0]
    kw = u.shape[1]
    return pl.pallas_call(
        functools.partial(_adjmm_kernel, rownorm_half=rownorm_half),
        grid=(n // bm,),
        in_specs=[
            pl.BlockSpec((bm, n), lambda i: (i, 0)),
            pl.BlockSpec((n, kw), lambda i: (0, 0)),
            pl.BlockSpec((1, kw), lambda i: (0, 0)),
        ],
        out_specs=pl.BlockSpec((bm, kw), lambda i: (i, 0)),
        out_shape=jax.ShapeDtypeStruct((n, kw), jnp.float32),
    )(adj, u, b)


def _dense_kernel(x_ref, w_ref, o_ref):
    o_ref[...] = jnp.dot(x_ref[...], w_ref[...], preferred_element_type=jnp.float32)


def _dense(x, w):
    n, f = x.shape
    kw = w.shape[1]
    return pl.pallas_call(
        _dense_kernel,
        out_shape=jax.ShapeDtypeStruct((n, kw), jnp.float32),
    )(x, w)


def _pairnorm_tanh(p):
    n, w = p.shape
    nh = w // 2
    m = jnp.mean(p, axis=0, keepdims=True)
    c = p - m
    ss = jnp.sum(c * c, axis=0, keepdims=True)
    s1 = jnp.sum(ss[:, :nh], axis=1, keepdims=True)
    s2 = jnp.sum(ss[:, nh:], axis=1, keepdims=True)
    inv1 = jax.lax.rsqrt(1e-6 + s1 / n)
    inv2 = jax.lax.rsqrt(1e-6 + s2 / n)
    scale = jnp.concatenate(
        [jnp.broadcast_to(inv1, (1, nh)), jnp.broadcast_to(inv2, (1, nh))], axis=1)
    return jnp.tanh(c * scale)


def _apply1_kernel(p_ref, x_ref, wx_ref, wh_ref, h_ref, u_ref):
    h = _pairnorm_tanh(p_ref[...])
    h_ref[...] = h
    u_ref[...] = (
        jnp.dot(x_ref[...], wx_ref[...], preferred_element_type=jnp.float32)
        + jnp.dot(h, wh_ref[...], preferred_element_type=jnp.float32))


def _apply1(p, x, wx, wh):
    n = p.shape[0]
    kw = wx.shape[1]
    return pl.pallas_call(
        _apply1_kernel,
        out_shape=(
            jax.ShapeDtypeStruct((n, p.shape[1]), jnp.float32),
            jax.ShapeDtypeStruct((n, kw), jnp.float32),
        ),
    )(p, x, wx, wh)


def _apply2_kernel(p_ref, x_ref, h0_ref, wx_ref, wh0_ref, wh1_ref, u_ref):
    h1 = _pairnorm_tanh(p_ref[...])
    u_ref[...] = (
        jnp.dot(x_ref[...], wx_ref[...], preferred_element_type=jnp.float32)
        + jnp.dot(h0_ref[...], wh0_ref[...], preferred_element_type=jnp.float32)
        + jnp.dot(h1, wh1_ref[...], preferred_element_type=jnp.float32))


def _apply2(p, x, h0, wx, wh0, wh1):
    n = p.shape[0]
    kw = wx.shape[1]
    return pl.pallas_call(
        _apply2_kernel,
        out_shape=jax.ShapeDtypeStruct((n, kw), jnp.float32),
    )(p, x, h0, wx, wh0, wh1)


def _attn_kernel(e1_ref, e2_ref, e3_ref, c1_ref, c2_ref, c3_ref,
                 wa1_ref, ba1_ref, wa2t_ref, wm_ref, bm_ref,
                 out_ref, beta_ref):
    e1, e2, e3 = e1_ref[...], e2_ref[...], e3_ref[...]
    xc = (c1_ref[...] + c2_ref[...] + c3_ref[...]) * (1.0 / 3.0)
    wa1, ba1, wa2t = wa1_ref[...], ba1_ref[...], wa2t_ref[...]
    ws = []
    for zb in (e1, e2, e3, xc):
        t = jnp.tanh(jnp.dot(zb, wa1, preferred_element_type=jnp.float32) + ba1)
        ws.append(jnp.sum(t * wa2t, axis=1, keepdims=True))
    w = jnp.concatenate(ws, axis=1)
    w = w - jnp.max(w, axis=1, keepdims=True)
    ew = jnp.exp(w)
    beta = ew / jnp.sum(ew, axis=1, keepdims=True)
    beta_ref[...] = beta
    emb = (beta[:, 0:1] * e1 + beta[:, 1:2] * e2
           + beta[:, 2:3] * e3 + beta[:, 3:4] * xc)
    logits = jnp.dot(emb, wm_ref[...], preferred_element_type=jnp.float32) + bm_ref[...]
    logits = logits - jnp.max(logits, axis=1, keepdims=True)
    el = jnp.exp(logits)
    out_ref[...] = el / jnp.sum(el, axis=1, keepdims=True)


def _attention(e1, e2, e3, c1, c2, c3, wa1, ba1, wa2, wm, bm, bm_rows=2000):
    n, d = e1.shape
    nclass = wm.shape[1]
    row = lambda i: (i, 0)
    const = lambda i: (0, 0)
    return pl.pallas_call(
        _attn_kernel,
        grid=(n // bm_rows,),
        in_specs=[pl.BlockSpec((bm_rows, d), row)] * 6 + [
            pl.BlockSpec(wa1.shape, const),
            pl.BlockSpec((1, ba1.shape[0]), const),
            pl.BlockSpec((1, wa2.shape[0]), const),
            pl.BlockSpec(wm.shape, const),
            pl.BlockSpec((1, bm.shape[0]), const),
        ],
        out_specs=(
            pl.BlockSpec((bm_rows, nclass), row),
            pl.BlockSpec((bm_rows, 4), row),
        ),
        out_shape=(
            jax.ShapeDtypeStruct((n, nclass), jnp.float32),
            jax.ShapeDtypeStruct((n, 4), jnp.float32),
        ),
    )(e1, e2, e3, c1, c2, c3, wa1, ba1.reshape(1, -1), wa2.reshape(1, -1),
      wm, bm.reshape(1, -1))


def _fused_pair(adj, x, pa, pb):
    f = x.shape[1]
    nh = pa["W0"].shape[1]
    out = pa["Wo"].shape[1]
    z = lambda r, c: jnp.zeros((r, c), jnp.float32)

    w0 = jnp.concatenate([pa["W0"], pb["W0"]], axis=1)
    b0 = jnp.concatenate([pa["b0"], pb["b0"]]).reshape(1, -1)
    w1x = jnp.concatenate([pa["W1"][:f], pb["W1"][:f]], axis=1)
    w1h = jnp.concatenate([
        jnp.concatenate([pa["W1"][f:], z(nh, nh)], axis=1),
        jnp.concatenate([z(nh, nh), pb["W1"][f:]], axis=1)], axis=0)
    b1 = jnp.concatenate([pa["b1"], pb["b1"]]).reshape(1, -1)
    wox = jnp.concatenate([pa["Wo"][:f], pb["Wo"][:f]], axis=1)
    woh0 = jnp.concatenate([
        jnp.concatenate([pa["Wo"][f:f + nh], z(nh, out)], axis=1),
        jnp.concatenate([z(nh, out), pb["Wo"][f:f + nh]], axis=1)], axis=0)
    woh1 = jnp.concatenate([
        jnp.concatenate([pa["Wo"][f + nh:], z(nh, out)], axis=1),
        jnp.concatenate([z(nh, out), pb["Wo"][f + nh:]], axis=1)], axis=0)
    bo = jnp.concatenate([pa["bo"], pb["bo"]]).reshape(1, -1)

    u0 = _dense(x, w0)
    p0 = _adjmm(adj, u0, b0)
    h0, u1 = _apply1(p0, x, w1x, w1h)
    p1 = _adjmm(adj, u1, b1)
    u2 = _apply2(p1, x, h0, wox, woh0, woh1)
    ec = _adjmm(adj, u2, bo, rownorm_half=True)
    return ec[:, :out], ec[:, out:]


def kernel(x, sadj, fadj, fadj2, params):
    x0, x1 = x[0], x[1]
    emb1, com1 = _fused_pair(sadj, x1, params["sg1"], params["cg"])
    emb2, com2 = _fused_pair(fadj, x0, params["sg2"], params["cg"])
    emb3, com3 = _fused_pair(fadj2, x0, params["sg3"], params["cg"])
    output, beta = _attention(
        emb1, emb2, emb3, com1, com2, com3,
        params["Wa1"], params["ba1"], params["Wa2"], params["Wm"], params["bm"])
    return (output, beta[..., None], emb1, com1, com2, com3, emb2, emb3)


# speedup vs baseline: 1.9790x; 1.1247x over previous
import jax
import jax.numpy as jnp
from jax.experimental import pallas as pl
from jax.experimental.pallas import tpu as pltpu

_BM0 = 200
_BM = 1000


def _pairnorm_tanh(p):
    n, w = p.shape
    nh = w // 2
    m = jnp.mean(p, axis=0, keepdims=True)
    c = p - m
    ss = jnp.sum(c * c, axis=0, keepdims=True)
    s1 = jnp.sum(ss[:, :nh], axis=1, keepdims=True)
    s2 = jnp.sum(ss[:, nh:], axis=1, keepdims=True)
    inv1 = jax.lax.rsqrt(1e-6 + s1 / n)
    inv2 = jax.lax.rsqrt(1e-6 + s2 / n)
    scale = jnp.concatenate(
        [jnp.broadcast_to(inv1, (1, nh)), jnp.broadcast_to(inv2, (1, nh))], axis=1)
    return jnp.tanh(c * scale)


def _dotf(a, b):
    return jnp.dot(a, b, preferred_element_type=jnp.float32)


_VMEM_CAP = pltpu.CompilerParams(vmem_limit_bytes=63 * 1024 * 1024)


def _layer0_kernel(a_ref, x_ref, w0_ref, b0_ref, w1x_ref, w1h_ref,
                   wox_ref, woh0_ref, abf_ref, u1_ref, uxh_ref,
                   u0_scr, p0_scr):
    i = pl.program_id(0)

    @pl.when(i == 0)
    def _():
        u0_scr[...] = _dotf(x_ref[...], w0_ref[...])

    a = a_ref[...]
    abf_ref[...] = a.astype(jnp.bfloat16)
    bm = a.shape[0]
    p0_scr[pl.ds(i * bm, bm), :] = _dotf(a, u0_scr[...]) + b0_ref[...]

    @pl.when(i == pl.num_programs(0) - 1)
    def _():
        h0 = _pairnorm_tanh(p0_scr[...])
        u1 = _dotf(x_ref[...], w1x_ref[...]) + _dotf(h0, w1h_ref[...])
        u1_ref[...] = u1.astype(jnp.bfloat16)
        uxh = _dotf(x_ref[...], wox_ref[...]) + _dotf(h0, woh0_ref[...])
        uxh_ref[...] = uxh.astype(jnp.bfloat16)


def _layer0(adj, x, w0, b0, w1x, w1h, wox, woh0):
    n, f = x.shape
    kw = w0.shape[1]
    ko = wox.shape[1]
    return pl.pallas_call(
        _layer0_kernel,
        grid=(n // _BM0,),
        in_specs=[
            pl.BlockSpec((_BM0, n), lambda i: (i, 0)),
            pl.BlockSpec((n, f), lambda i: (0, 0)),
            pl.BlockSpec((f, kw), lambda i: (0, 0)),
            pl.BlockSpec((1, kw), lambda i: (0, 0)),
            pl.BlockSpec((f, kw), lambda i: (0, 0)),
            pl.BlockSpec((kw, kw), lambda i: (0, 0)),
            pl.BlockSpec((f, ko), lambda i: (0, 0)),
            pl.BlockSpec((kw, ko), lambda i: (0, 0)),
        ],
        out_specs=(
            pl.BlockSpec((_BM0, n), lambda i: (i, 0)),
            pl.BlockSpec((n, kw), lambda i: (0, 0)),
            pl.BlockSpec((n, ko), lambda i: (0, 0)),
        ),
        out_shape=(
            jax.ShapeDtypeStruct((n, n), jnp.bfloat16),
            jax.ShapeDtypeStruct((n, kw), jnp.bfloat16),
            jax.ShapeDtypeStruct((n, ko), jnp.bfloat16),
        ),
        scratch_shapes=[pltpu.VMEM((n, kw), jnp.float32),
                        pltpu.VMEM((n, kw), jnp.float32)],
        compiler_params=_VMEM_CAP,
    )(adj, x, w0, b0, w1x, w1h, wox, woh0)


def _layer1_kernel(abf_ref, u1_ref, b1_ref, uxh_ref, woh1_ref,
                   u2_ref, p1_scr):
    i = pl.program_id(0)
    a = abf_ref[...]
    bm = a.shape[0]
    p1_scr[pl.ds(i * bm, bm), :] = _dotf(a, u1_ref[...]) + b1_ref[...]

    @pl.when(i == pl.num_programs(0) - 1)
    def _():
        h1 = _pairnorm_tanh(p1_scr[...])
        u2 = uxh_ref[...].astype(jnp.float32) + _dotf(h1, woh1_ref[...])
        u2_ref[...] = u2.astype(jnp.bfloat16)


def _layer1(abf, u1, b1, uxh, woh1):
    n = abf.shape[0]
    kh = u1.shape[1]
    kw = uxh.shape[1]
    return pl.pallas_call(
        _layer1_kernel,
        grid=(n // _BM,),
        in_specs=[
            pl.BlockSpec((_BM, n), lambda i: (i, 0)),
            pl.BlockSpec((n, kh), lambda i: (0, 0)),
            pl.BlockSpec((1, kh), lambda i: (0, 0)),
            pl.BlockSpec((n, kw), lambda i: (0, 0)),
            pl.BlockSpec((kh, kw), lambda i: (0, 0)),
        ],
        out_specs=pl.BlockSpec((n, kw), lambda i: (0, 0)),
        out_shape=jax.ShapeDtypeStruct((n, kw), jnp.bfloat16),
        scratch_shapes=[pltpu.VMEM((n, kh), jnp.float32)],
        compiler_params=_VMEM_CAP,
    )(abf, u1, b1, uxh, woh1)


def _rownorm_halves(p):
    h = p.shape[-1] // 2
    p1, p2 = p[:, :h], p[:, h:]
    i1 = jax.lax.rsqrt(jnp.maximum(jnp.sum(p1 * p1, axis=1, keepdims=True), 1e-24))
    i2 = jax.lax.rsqrt(jnp.maximum(jnp.sum(p2 * p2, axis=1, keepdims=True), 1e-24))
    return p1 * i1, p2 * i2


def _layer2_kernel(abf_ref, u2_ref, bo_ref, ec_ref):
    p = _dotf(abf_ref[...], u2_ref[...]) + bo_ref[...]
    ea, eb = _rownorm_halves(p)
    ec_ref[...] = jnp.concatenate([ea, eb], axis=1)


def _layer2(abf, u2, bo):
    n = abf.shape[0]
    kw = u2.shape[1]
    return pl.pallas_call(
        _layer2_kernel,
        grid=(n // _BM,),
        in_specs=[
            pl.BlockSpec((_BM, n), lambda i: (i, 0)),
            pl.BlockSpec((n, kw), lambda i: (0, 0)),
            pl.BlockSpec((1, kw), lambda i: (0, 0)),
        ],
        out_specs=pl.BlockSpec((_BM, kw), lambda i: (i, 0)),
        out_shape=jax.ShapeDtypeStruct((n, kw), jnp.float32),
        compiler_params=_VMEM_CAP,
    )(abf, u2, bo)


def _layer2_attn_kernel(abf_ref, u2_ref, bo_ref, e1_ref, e2_ref, c1_ref,
                        c2_ref, wa1_ref, ba1_ref, wa2t_ref, wm_ref, bm_ref,
                        ec_ref, out_ref, beta_ref):
    p = _dotf(abf_ref[...], u2_ref[...]) + bo_ref[...]
    e3, c3 = _rownorm_halves(p)
    ec_ref[...] = jnp.concatenate([e3, c3], axis=1)
    e1, e2 = e1_ref[...], e2_ref[...]
    xc = (c1_ref[...] + c2_ref[...] + c3) * (1.0 / 3.0)
    wa1, ba1, wa2t = wa1_ref[...], ba1_ref[...], wa2t_ref[...]
    ws = []
    for zb in (e1, e2, e3, xc):
        t = jnp.tanh(_dotf(zb, wa1) + ba1)
        ws.append(jnp.sum(t * wa2t, axis=1, keepdims=True))
    w = jnp.concatenate(ws, axis=1)
    w = w - jnp.max(w, axis=1, keepdims=True)
    ew = jnp.exp(w)
    beta = ew / jnp.sum(ew, axis=1, keepdims=True)
    beta_ref[...] = beta
    emb = (beta[:, 0:1] * e1 + beta[:, 1:2] * e2
           + beta[:, 2:3] * e3 + beta[:, 3:4] * xc)
    logits = _dotf(emb, wm_ref[...]) + bm_ref[...]
    logits = logits - jnp.max(logits, axis=1, keepdims=True)
    el = jnp.exp(logits)
    out_ref[...] = el / jnp.sum(el, axis=1, keepdims=True)


def _layer2_attn(abf, u2, bo, e1, e2, c1, c2, wa1, ba1, wa2, wm, bm):
    n = abf.shape[0]
    kw = u2.shape[1]
    d = e1.shape[1]
    nclass = wm.shape[1]
    row = lambda i: (i, 0)
    const = lambda i: (0, 0)
    return pl.pallas_call(
        _layer2_attn_kernel,
        grid=(n // _BM,),
        in_specs=[
            pl.BlockSpec((_BM, n), row),
            pl.BlockSpec((n, kw), const),
            pl.BlockSpec((1, kw), const),
            pl.BlockSpec((_BM, d), row),
            pl.BlockSpec((_BM, d), row),
            pl.BlockSpec((_BM, d), row),
            pl.BlockSpec((_BM, d), row),
            pl.BlockSpec(wa1.shape, const),
            pl.BlockSpec((1, ba1.shape[0]), const),
            pl.BlockSpec((1, wa2.shape[0]), const),
            pl.BlockSpec(wm.shape, const),
            pl.BlockSpec((1, bm.shape[0]), const),
        ],
        out_specs=(
            pl.BlockSpec((_BM, kw), row),
            pl.BlockSpec((_BM, nclass), row),
            pl.BlockSpec((_BM, 4), row),
        ),
        out_shape=(
            jax.ShapeDtypeStruct((n, kw), jnp.float32),
            jax.ShapeDtypeStruct((n, nclass), jnp.float32),
            jax.ShapeDtypeStruct((n, 4), jnp.float32),
        ),
        compiler_params=_VMEM_CAP,
    )(abf, u2, bo, e1, e2, c1, c2,
      wa1, ba1.reshape(1, -1), wa2.reshape(1, -1), wm, bm.reshape(1, -1))


def _pair_weights(x, pa, pb):
    f = x.shape[1]
    nh = pa["W0"].shape[1]
    out = pa["Wo"].shape[1]
    z = lambda r, c: jnp.zeros((r, c), jnp.float32)

    w0 = jnp.concatenate([pa["W0"], pb["W0"]], axis=1)
    b0 = jnp.concatenate([pa["b0"], pb["b0"]]).reshape(1, -1)
    w1x = jnp.concatenate([pa["W1"][:f], pb["W1"][:f]], axis=1)
    w1h = jnp.concatenate([
        jnp.concatenate([pa["W1"][f:], z(nh, nh)], axis=1),
        jnp.concatenate([z(nh, nh), pb["W1"][f:]], axis=1)], axis=0)
    b1 = jnp.concatenate([pa["b1"], pb["b1"]]).reshape(1, -1)
    wox = jnp.concatenate([pa["Wo"][:f], pb["Wo"][:f]], axis=1)
    woh0 = jnp.concatenate([
        jnp.concatenate([pa["Wo"][f:f + nh], z(nh, out)], axis=1),
        jnp.concatenate([z(nh, out), pb["Wo"][f:f + nh]], axis=1)], axis=0)
    woh1 = jnp.concatenate([
        jnp.concatenate([pa["Wo"][f + nh:], z(nh, out)], axis=1),
        jnp.concatenate([z(nh, out), pb["Wo"][f + nh:]], axis=1)], axis=0)
    bo = jnp.concatenate([pa["bo"], pb["bo"]]).reshape(1, -1)
    return w0, b0, w1x, w1h, b1, wox, woh0, woh1, bo


def _pair_u2(adj, x, pa, pb):
    w0, b0, w1x, w1h, b1, wox, woh0, woh1, bo = _pair_weights(x, pa, pb)
    abf, u1, uxh = _layer0(adj, x, w0, b0, w1x, w1h, wox, woh0)
    u2 = _layer1(abf, u1, b1, uxh, woh1)
    return abf, u2, bo


def kernel(x, sadj, fadj, fadj2, params):
    x0, x1 = x[0], x[1]
    out = params["sg1"]["Wo"].shape[1]

    abf1, u2_1, bo1 = _pair_u2(sadj, x1, params["sg1"], params["cg"])
    ec1 = _layer2(abf1, u2_1, bo1)
    emb1, com1 = ec1[:, :out], ec1[:, out:]

    abf2, u2_2, bo2 = _pair_u2(fadj, x0, params["sg2"], params["cg"])
    ec2 = _layer2(abf2, u2_2, bo2)
    emb2, com2 = ec2[:, :out], ec2[:, out:]

    abf3, u2_3, bo3 = _pair_u2(fadj2, x0, params["sg3"], params["cg"])
    ec3, output, beta = _layer2_attn(
        abf3, u2_3, bo3, emb1, emb2, com1, com2,
        params["Wa1"], params["ba1"], params["Wa2"], params["Wm"], params["bm"])
    emb3, com3 = ec3[:, :out], ec3[:, out:]
    return (output, beta[..., None], emb1, com1, com2, com3, emb2, emb3)


# speedup vs baseline: 2.0351x; 1.0283x over previous
import jax
import jax.numpy as jnp
from jax.experimental import pallas as pl
from jax.experimental.pallas import tpu as pltpu

_BM0 = 200
_BM = 1000


def _pairnorm_tanh(p):
    n, w = p.shape
    nh = w // 2
    m = jnp.mean(p, axis=0, keepdims=True)
    c = p - m
    ss = jnp.sum(c * c, axis=0, keepdims=True)
    s1 = jnp.sum(ss[:, :nh], axis=1, keepdims=True)
    s2 = jnp.sum(ss[:, nh:], axis=1, keepdims=True)
    inv1 = jax.lax.rsqrt(1e-6 + s1 / n)
    inv2 = jax.lax.rsqrt(1e-6 + s2 / n)
    scale = jnp.concatenate(
        [jnp.broadcast_to(inv1, (1, nh)), jnp.broadcast_to(inv2, (1, nh))], axis=1)
    return jnp.tanh(c * scale)


def _dotf(a, b):
    return jnp.dot(a, b, preferred_element_type=jnp.float32)


def _dotf_split(a, b, parts=4):
    k = a.shape[1]
    step = k // parts
    acc = None
    for g in range(parts):
        part = jnp.dot(a[:, g * step:(g + 1) * step],
                       b[g * step:(g + 1) * step, :],
                       preferred_element_type=jnp.float32)
        acc = part if acc is None else acc + part
    return acc


_VMEM_CAP = pltpu.CompilerParams(vmem_limit_bytes=63 * 1024 * 1024)


def _layer0_kernel(a_ref, x_ref, w0_ref, b0_ref, w1x_ref, w1h_ref,
                   wox_ref, woh0_ref, abf_ref, u1_ref, uxh_ref,
                   u0_scr, p0_scr):
    i = pl.program_id(0)

    @pl.when(i == 0)
    def _():
        u0_scr[...] = _dotf(x_ref[...], w0_ref[...])

    a = a_ref[...]
    abf_ref[...] = a.astype(jnp.bfloat16)
    bm = a.shape[0]
    p0_scr[pl.ds(i * bm, bm), :] = _dotf(a, u0_scr[...]) + b0_ref[...]

    @pl.when(i == pl.num_programs(0) - 1)
    def _():
        h0 = _pairnorm_tanh(p0_scr[...])
        u1 = _dotf(x_ref[...], w1x_ref[...]) + _dotf(h0, w1h_ref[...])
        u1_ref[...] = u1.astype(jnp.bfloat16)
        uxh = _dotf(x_ref[...], wox_ref[...]) + _dotf(h0, woh0_ref[...])
        uxh_ref[...] = uxh.astype(jnp.bfloat16)


def _layer0(adj, x, w0, b0, w1x, w1h, wox, woh0):
    n, f = x.shape
    kw = w0.shape[1]
    ko = wox.shape[1]
    return pl.pallas_call(
        _layer0_kernel,
        grid=(n // _BM0,),
        in_specs=[
            pl.BlockSpec((_BM0, n), lambda i: (i, 0)),
            pl.BlockSpec((n, f), lambda i: (0, 0)),
            pl.BlockSpec((f, kw), lambda i: (0, 0)),
            pl.BlockSpec((1, kw), lambda i: (0, 0)),
            pl.BlockSpec((f, kw), lambda i: (0, 0)),
            pl.BlockSpec((kw, kw), lambda i: (0, 0)),
            pl.BlockSpec((f, ko), lambda i: (0, 0)),
            pl.BlockSpec((kw, ko), lambda i: (0, 0)),
        ],
        out_specs=(
            pl.BlockSpec((_BM0, n), lambda i: (i, 0)),
            pl.BlockSpec((n, kw), lambda i: (0, 0)),
            pl.BlockSpec((n, ko), lambda i: (0, 0)),
        ),
        out_shape=(
            jax.ShapeDtypeStruct((n, n), jnp.bfloat16),
            jax.ShapeDtypeStruct((n, kw), jnp.bfloat16),
            jax.ShapeDtypeStruct((n, ko), jnp.bfloat16),
        ),
        scratch_shapes=[pltpu.VMEM((n, kw), jnp.float32),
                        pltpu.VMEM((n, kw), jnp.float32)],
        compiler_params=_VMEM_CAP,
    )(adj, x, w0, b0, w1x, w1h, wox, woh0)


def _layer1_kernel(abf_ref, u1_ref, b1_ref, uxh_ref, woh1_ref,
                   u2_ref, p1_scr):
    i = pl.program_id(0)
    a = abf_ref[...]
    bm = a.shape[0]
    p1_scr[pl.ds(i * bm, bm), :] = _dotf(a, u1_ref[...]) + b1_ref[...]

    @pl.when(i == pl.num_programs(0) - 1)
    def _():
        h1 = _pairnorm_tanh(p1_scr[...])
        u2 = uxh_ref[...].astype(jnp.float32) + _dotf(h1, woh1_ref[...])
        u2_ref[...] = u2.astype(jnp.bfloat16)


def _layer1(abf, u1, b1, uxh, woh1):
    n = abf.shape[0]
    kh = u1.shape[1]
    kw = uxh.shape[1]
    return pl.pallas_call(
        _layer1_kernel,
        grid=(n // _BM,),
        in_specs=[
            pl.BlockSpec((_BM, n), lambda i: (i, 0)),
            pl.BlockSpec((n, kh), lambda i: (0, 0)),
            pl.BlockSpec((1, kh), lambda i: (0, 0)),
            pl.BlockSpec((n, kw), lambda i: (0, 0)),
            pl.BlockSpec((kh, kw), lambda i: (0, 0)),
        ],
        out_specs=pl.BlockSpec((n, kw), lambda i: (0, 0)),
        out_shape=jax.ShapeDtypeStruct((n, kw), jnp.bfloat16),
        scratch_shapes=[pltpu.VMEM((n, kh), jnp.float32)],
        compiler_params=_VMEM_CAP,
    )(abf, u1, b1, uxh, woh1)


def _rownorm_halves(p):
    h = p.shape[-1] // 2
    p1, p2 = p[:, :h], p[:, h:]
    i1 = jax.lax.rsqrt(jnp.maximum(jnp.sum(p1 * p1, axis=1, keepdims=True), 1e-24))
    i2 = jax.lax.rsqrt(jnp.maximum(jnp.sum(p2 * p2, axis=1, keepdims=True), 1e-24))
    return p1 * i1, p2 * i2


def _layer2_kernel(abf_ref, u2_ref, bo_ref, ec_ref):
    p = _dotf(abf_ref[...], u2_ref[...]) + bo_ref[...]
    ea, eb = _rownorm_halves(p)
    ec_ref[...] = jnp.concatenate([ea, eb], axis=1)


def _layer2(abf, u2, bo):
    n = abf.shape[0]
    kw = u2.shape[1]
    return pl.pallas_call(
        _layer2_kernel,
        grid=(n // _BM,),
        in_specs=[
            pl.BlockSpec((_BM, n), lambda i: (i, 0)),
            pl.BlockSpec((n, kw), lambda i: (0, 0)),
            pl.BlockSpec((1, kw), lambda i: (0, 0)),
        ],
        out_specs=pl.BlockSpec((_BM, kw), lambda i: (i, 0)),
        out_shape=jax.ShapeDtypeStruct((n, kw), jnp.float32),
        compiler_params=_VMEM_CAP,
    )(abf, u2, bo)


def _attn_kernel(e1_ref, e2_ref, e3_ref, c1_ref, c2_ref, c3_ref,
                 wa1_ref, ba1_ref, wa2t_ref, wm_ref, bm_ref,
                 out_ref, beta_ref):
    e1, e2, e3 = e1_ref[...], e2_ref[...], e3_ref[...]
    xc = (c1_ref[...] + c2_ref[...] + c3_ref[...]) * (1.0 / 3.0)
    wa1, ba1, wa2t = wa1_ref[...], ba1_ref[...], wa2t_ref[...]
    ws = []
    for zb in (e1, e2, e3, xc):
        t = jnp.tanh(_dotf(zb, wa1) + ba1)
        ws.append(jnp.sum(t * wa2t, axis=1, keepdims=True))
    w = jnp.concatenate(ws, axis=1)
    w = w - jnp.max(w, axis=1, keepdims=True)
    ew = jnp.exp(w)
    beta = ew / jnp.sum(ew, axis=1, keepdims=True)
    beta_ref[...] = beta
    emb = (beta[:, 0:1] * e1 + beta[:, 1:2] * e2
           + beta[:, 2:3] * e3 + beta[:, 3:4] * xc)
    logits = _dotf(emb, wm_ref[...]) + bm_ref[...]
    logits = logits - jnp.max(logits, axis=1, keepdims=True)
    el = jnp.exp(logits)
    out_ref[...] = el / jnp.sum(el, axis=1, keepdims=True)


def _attention(e1, e2, e3, c1, c2, c3, wa1, ba1, wa2, wm, bm, bm_rows=2000):
    n, d = e1.shape
    nclass = wm.shape[1]
    row = lambda i: (i, 0)
    const = lambda i: (0, 0)
    return pl.pallas_call(
        _attn_kernel,
        grid=(n // bm_rows,),
        in_specs=[pl.BlockSpec((bm_rows, d), row)] * 6 + [
            pl.BlockSpec(wa1.shape, const),
            pl.BlockSpec((1, ba1.shape[0]), const),
            pl.BlockSpec((1, wa2.shape[0]), const),
            pl.BlockSpec(wm.shape, const),
            pl.BlockSpec((1, bm.shape[0]), const),
        ],
        out_specs=(
            pl.BlockSpec((bm_rows, nclass), row),
            pl.BlockSpec((bm_rows, 4), row),
        ),
        out_shape=(
            jax.ShapeDtypeStruct((n, nclass), jnp.float32),
            jax.ShapeDtypeStruct((n, 4), jnp.float32),
        ),
    )(e1, e2, e3, c1, c2, c3, wa1, ba1.reshape(1, -1), wa2.reshape(1, -1),
      wm, bm.reshape(1, -1))


def _pair_weights(x, pa, pb):
    f = x.shape[1]
    nh = pa["W0"].shape[1]
    out = pa["Wo"].shape[1]
    z = lambda r, c: jnp.zeros((r, c), jnp.float32)

    w0 = jnp.concatenate([pa["W0"], pb["W0"]], axis=1)
    b0 = jnp.concatenate([pa["b0"], pb["b0"]]).reshape(1, -1)
    w1x = jnp.concatenate([pa["W1"][:f], pb["W1"][:f]], axis=1)
    w1h = jnp.concatenate([
        jnp.concatenate([pa["W1"][f:], z(nh, nh)], axis=1),
        jnp.concatenate([z(nh, nh), pb["W1"][f:]], axis=1)], axis=0)
    b1 = jnp.concatenate([pa["b1"], pb["b1"]]).reshape(1, -1)
    wox = jnp.concatenate([pa["Wo"][:f], pb["Wo"][:f]], axis=1)
    woh0 = jnp.concatenate([
        jnp.concatenate([pa["Wo"][f:f + nh], z(nh, out)], axis=1),
        jnp.concatenate([z(nh, out), pb["Wo"][f:f + nh]], axis=1)], axis=0)
    woh1 = jnp.concatenate([
        jnp.concatenate([pa["Wo"][f + nh:], z(nh, out)], axis=1),
        jnp.concatenate([z(nh, out), pb["Wo"][f + nh:]], axis=1)], axis=0)
    bo = jnp.concatenate([pa["bo"], pb["bo"]]).reshape(1, -1)
    return w0, b0, w1x, w1h, b1, wox, woh0, woh1, bo


def _pair_u2(adj, x, pa, pb):
    w0, b0, w1x, w1h, b1, wox, woh0, woh1, bo = _pair_weights(x, pa, pb)
    abf, u1, uxh = _layer0(adj, x, w0, b0, w1x, w1h, wox, woh0)
    u2 = _layer1(abf, u1, b1, uxh, woh1)
    return abf, u2, bo


def kernel(x, sadj, fadj, fadj2, params):
    x0, x1 = x[0], x[1]
    out = params["sg1"]["Wo"].shape[1]

    abf1, u2_1, bo1 = _pair_u2(sadj, x1, params["sg1"], params["cg"])
    ec1 = _layer2(abf1, u2_1, bo1)
    emb1, com1 = ec1[:, :out], ec1[:, out:]

    abf2, u2_2, bo2 = _pair_u2(fadj, x0, params["sg2"], params["cg"])
    ec2 = _layer2(abf2, u2_2, bo2)
    emb2, com2 = ec2[:, :out], ec2[:, out:]

    abf3, u2_3, bo3 = _pair_u2(fadj2, x0, params["sg3"], params["cg"])
    ec3 = _layer2(abf3, u2_3, bo3)
    emb3, com3 = ec3[:, :out], ec3[:, out:]

    output, beta = _attention(
        emb1, emb2, emb3, com1, com2, com3,
        params["Wa1"], params["ba1"], params["Wa2"], params["Wm"], params["bm"])
    return (output, beta[..., None], emb1, com1, com2, com3, emb2, emb3)
